# baseline (speedup 1.0000x reference)
import functools

import jax
import jax.numpy as jnp
from jax import lax
from jax.experimental import pallas as pl
from jax.experimental.pallas import tpu as pltpu
from jax.experimental.pallas import tpu_sc as plsc

_BN_EPS = 1e-5


def _sc_dims():
    try:
        info = plsc.get_sparse_core_info()
        return info.num_cores, info.num_subcores
    except Exception:
        return 2, 16


def _make_sc_gather(n, h, e, nc, ns, s, csteps):
    nw = nc * ns
    per_w = e // nw
    steps = per_w // s
    chunks = steps // csteps
    nbuf = 5
    assert chunks * csteps == steps and csteps % nbuf == 0
    mesh = plsc.VectorSubcoreMesh(core_axis_name="c", subcore_axis_name="s")

    @functools.partial(
        pl.kernel,
        mesh=mesh,
        out_type=jax.ShapeDtypeStruct((e, h), jnp.float32),
        scratch_types=(
            [pltpu.VMEM((csteps, s), jnp.int32)] * 2
            + [pltpu.VMEM((s, h), jnp.float32)] * nbuf
            + [pltpu.SemaphoreType.DMA] * (2 * nbuf)
        ),
    )
    def gather_kernel(node_hbm, src_hbm, dst_hbm, out_hbm, src_v, dst_v,
                      *bufs):
        rows = bufs[:nbuf]
        sg = bufs[nbuf:2 * nbuf]
        ss = bufs[2 * nbuf:]
        wid = lax.axis_index("s") * nc + lax.axis_index("c")
        base = wid * per_w

        def drain_store(b):
            pltpu.make_async_copy(rows[b], out_hbm.at[pl.ds(base, s)],
                                  ss[b]).wait()

        def chunk_body(ch, carry):
            pltpu.sync_copy(src_hbm.at[wid, ch], src_v)
            pltpu.sync_copy(dst_hbm.at[wid, ch], dst_v)
            cbase = ch * csteps

            def body(t, carry2):
                i = nbuf * t
                cps = []
                for b in range(nbuf):
                    @pl.when(t > 0)
                    def _(b=b):
                        drain_store(b)

                    cps.append(pltpu.async_copy(
                        node_hbm.at[src_v.at[i + b]], rows[b], sg[b]))
                dps = []
                for b in range(nbuf):
                    cps[b].wait()
                    dps.append(pltpu.async_copy(
                        node_hbm.at[dst_v.at[i + b]], rows[b], sg[b],
                        add=True))
                for b in range(nbuf):
                    dps[b].wait()
                    pltpu.async_copy(
                        rows[b],
                        out_hbm.at[pl.ds(base + (cbase + i + b) * s, s)],
                        ss[b])
                return carry2

            lax.fori_loop(0, csteps // nbuf, body, 0)
            for b in range(nbuf):
                drain_store(b)
            return carry

        lax.fori_loop(0, chunks, chunk_body, 0)

    return gather_kernel


def _make_sc_scatter(n, h, e, nc, ns, s, csteps):
    nw = nc * ns
    per_w = e // nw
    steps = per_w // s
    chunks = steps // csteps
    assert chunks * csteps == steps and csteps % 2 == 0
    rpt = (n // ns) // 8 * 8
    tail = n - rpt * ns
    mesh = plsc.VectorSubcoreMesh(core_axis_name="c", subcore_axis_name="s")

    @functools.partial(
        pl.kernel,
        mesh=mesh,
        out_type=jax.ShapeDtypeStruct((nc, n, h), jnp.float32),
        scratch_types=[
            pltpu.VMEM((csteps, s), jnp.int32),
            pltpu.VMEM((csteps, s), jnp.int32),
            pltpu.VMEM((s, h), jnp.float32),
            pltpu.VMEM((s, h), jnp.float32),
            pltpu.VMEM_SHARED((n, h), jnp.float32),
            pltpu.SemaphoreType.DMA,
            pltpu.SemaphoreType.DMA,
            pltpu.SemaphoreType.DMA,
            pltpu.SemaphoreType.DMA,
        ],
    )
    def scatter_kernel(e2n_hbm, src_hbm, dst_hbm, zeros_hbm, out_hbm,
                       src_v, dst_v, r0, r1, agg_sh, lg0, lg1, sc0, sc1):
        cid = lax.axis_index("c")
        sid = lax.axis_index("s")
        wid = sid * nc + cid
        base = wid * per_w
        pltpu.sync_copy(zeros_hbm.at[pl.ds(sid * rpt, rpt)],
                        agg_sh.at[pl.ds(sid * rpt, rpt)])

        @pl.when(sid == ns - 1)
        def _():
            pltpu.sync_copy(zeros_hbm.at[pl.ds(rpt * ns, tail)],
                            agg_sh.at[pl.ds(rpt * ns, tail)])

        plsc.subcore_barrier()
        rows = (r0, r1)
        lg = (lg0, lg1)
        sc = (sc0, sc1)

        def load(gi, b):
            return pltpu.async_copy(e2n_hbm.at[pl.ds(base + gi * s, s)],
                                    rows[b], lg[b])

        def chunk_body(ch, carry):
            pltpu.sync_copy(src_hbm.at[wid, ch], src_v)
            pltpu.sync_copy(dst_hbm.at[wid, ch], dst_v)
            cbase = ch * csteps

            def body(t, carry2):
                i = 2 * t
                c0 = load(cbase + i, 0)
                c1 = load(cbase + i + 1, 1)
                c0.wait()
                s0a = pltpu.async_copy(rows[0], agg_sh.at[src_v.at[i]],
                                       sc[0], add=True)
                s0b = pltpu.async_copy(rows[0], agg_sh.at[dst_v.at[i]],
                                       sc[0], add=True)
                c1.wait()
                s1a = pltpu.async_copy(rows[1], agg_sh.at[src_v.at[i + 1]],
                                       sc[1], add=True)
                s1b = pltpu.async_copy(rows[1], agg_sh.at[dst_v.at[i + 1]],
                                       sc[1], add=True)
                s0a.wait()
                s0b.wait()
                s1a.wait()
                s1b.wait()
                return carry2

            lax.fori_loop(0, csteps // 2, body, 0)
            return carry

        lax.fori_loop(0, chunks, chunk_body, 0)
        plsc.subcore_barrier()
        pltpu.sync_copy(agg_sh.at[pl.ds(sid * rpt, rpt)],
                        out_hbm.at[cid, pl.ds(sid * rpt, rpt)])

        @pl.when(sid == ns - 1)
        def _():
            pltpu.sync_copy(agg_sh.at[pl.ds(rpt * ns, tail)],
                            out_hbm.at[cid, pl.ds(rpt * ns, tail)])

    return scatter_kernel


def _finalize_stats(st, count, g, b):
    mean = st[0] / count
    var = st[1] / count - mean * mean
    scale = g * jax.lax.rsqrt(var + _BN_EPS)
    shift = b - mean * scale
    return scale.reshape(1, -1), shift.reshape(1, -1)


def _accum_stats(stats_ref, y, j):
    @pl.when(j == 0)
    def _():
        stats_ref[...] = jnp.zeros_like(stats_ref)

    stats_ref[0:1, :] += jnp.sum(y, axis=0, keepdims=True)
    stats_ref[1:2, :] += jnp.sum(y * y, axis=0, keepdims=True)


def _p1_body(n2e_ref, er_ref, wt_top_ref, wt_bot_ref, y1_ref, st_ref):
    j = pl.program_id(0)
    y = (jnp.dot(n2e_ref[...].astype(jnp.bfloat16), wt_top_ref[...],
                 preferred_element_type=jnp.float32)
         + jnp.dot(er_ref[...].astype(jnp.bfloat16), wt_bot_ref[...],
                   preferred_element_type=jnp.float32))
    y1_ref[...] = y.astype(jnp.bfloat16)
    _accum_stats(st_ref, y, j)


def _p2_body(y1_ref, sc_ref, sh_ref, wt_ref, y2_ref, st_ref):
    j = pl.program_id(0)
    h = jnp.maximum(
        y1_ref[...].astype(jnp.float32) * sc_ref[...] + sh_ref[...], 0.0)
    y = jnp.dot(h.astype(jnp.bfloat16), wt_ref[...],
                preferred_element_type=jnp.float32)
    y2_ref[...] = y.astype(jnp.bfloat16)
    _accum_stats(st_ref, y, j)


def _p3_body(y2_ref, er_ref, sc_ref, sh_ref, c1_ref, wt1_ref, wt2_ref,
             y3_ref, y5_ref, st3_ref, st5_ref):
    j = pl.program_id(0)
    m = jnp.maximum(
        y2_ref[...].astype(jnp.float32) * sc_ref[...] + sh_ref[...], 0.0)
    en = (c1_ref[0, 0] * er_ref[...] + m).astype(jnp.bfloat16)
    y3 = jnp.dot(en, wt1_ref[...], preferred_element_type=jnp.float32)
    y5 = jnp.dot(en, wt2_ref[...], preferred_element_type=jnp.float32)
    y3_ref[...] = y3.astype(jnp.bfloat16)
    y5_ref[...] = y5.astype(jnp.bfloat16)
    _accum_stats(st3_ref, y3, j)
    _accum_stats(st5_ref, y5, j)


def _p4_body(y3_ref, y5_ref, sc3_ref, sh3_ref, sc5_ref, sh5_ref,
             wt1_ref, wt2_ref, y4_ref, y6_ref, st4_ref, st6_ref):
    j = pl.program_id(0)
    h3 = jnp.maximum(
        y3_ref[...].astype(jnp.float32) * sc3_ref[...] + sh3_ref[...], 0.0)
    h5 = jnp.maximum(
        y5_ref[...].astype(jnp.float32) * sc5_ref[...] + sh5_ref[...], 0.0)
    y4 = jnp.dot(h3.astype(jnp.bfloat16), wt1_ref[...],
                 preferred_element_type=jnp.float32)
    y6 = jnp.dot(h5.astype(jnp.bfloat16), wt2_ref[...],
                 preferred_element_type=jnp.float32)
    y4_ref[...] = y4.astype(jnp.bfloat16)
    y6_ref[...] = y6.astype(jnp.bfloat16)
    _accum_stats(st4_ref, y4, j)
    _accum_stats(st6_ref, y6, j)


def _p5_body(y4_ref, y6_ref, sc4_ref, sh4_ref, sc6_ref, sh6_ref,
             eo_ref, e2n_ref):
    eo_ref[...] = jnp.maximum(
        y4_ref[...].astype(jnp.float32) * sc4_ref[...] + sh4_ref[...], 0.0)
    e2n_ref[...] = jnp.maximum(
        y6_ref[...].astype(jnp.float32) * sc6_ref[...] + sh6_ref[...], 0.0)


def _node_body(nr_ref, coef_ref, agg0_ref, agg1_ref, wa_ref, ga_ref, ba_ref,
               wb_ref, gb_ref, bb_ref, out_ref):
    x = coef_ref[...] * nr_ref[...] + (agg0_ref[...] + agg1_ref[...])
    y = jnp.dot(x, wa_ref[...], preferred_element_type=jnp.float32)
    m = jnp.mean(y, axis=0, keepdims=True)
    v = jnp.mean(y * y, axis=0, keepdims=True) - m * m
    h = jnp.maximum((y - m) * jax.lax.rsqrt(v + _BN_EPS) * ga_ref[...]
                    + ba_ref[...], 0.0)
    y2 = jnp.dot(h, wb_ref[...], preferred_element_type=jnp.float32)
    m2 = jnp.mean(y2, axis=0, keepdims=True)
    v2 = jnp.mean(y2 * y2, axis=0, keepdims=True) - m2 * m2
    out_ref[...] = jnp.maximum(
        (y2 - m2) * jax.lax.rsqrt(v2 + _BN_EPS) * gb_ref[...] + bb_ref[...],
        0.0)


def _edge_block(be, c):
    return pl.BlockSpec((be, c), lambda j: (j, 0))


def _full_block(shape):
    return pl.BlockSpec(shape, lambda j: tuple(0 for _ in shape))


def _stats_spec(c):
    return pl.BlockSpec((8, c), lambda j: (0, 0))


def kernel(node_rep, edge_rep, edge_index, degree, W0a, g0a, b0a, W0b, g0b,
           b0b, W1a, g1a, b1a, W1b, g1b, b1b, W2a, g2a, b2a, W2b, g2b, b2b,
           Wna, gna, bna, Wnb, gnb, bnb, eps1, eps2):
    n, h = node_rep.shape
    e = edge_rep.shape[0]
    h2 = 2 * h
    be = 8000
    nb = e // be
    assert nb * be == e
    f32 = jnp.float32

    src = edge_index[0]
    dst = edge_index[1]
    nc, ns = _sc_dims()
    nw = nc * ns
    bf16 = jnp.bfloat16
    s_sc = 40
    c_sc = 50
    n_ch = e // nw // s_sc // c_sc
    src3 = src.reshape(nw, n_ch, c_sc, s_sc)
    dst3 = dst.reshape(nw, n_ch, c_sc, s_sc)
    n2e = _make_sc_gather(n, h, e, nc, ns, s_sc, c_sc)(node_rep, src3, dst3)

    grid = (nb,)

    wt0 = W0a.T.astype(bf16)
    y1, st1 = pl.pallas_call(
        _p1_body,
        grid=grid,
        in_specs=[_edge_block(be, h), _edge_block(be, h),
                  _full_block((h, h2)), _full_block((h, h2))],
        out_specs=[_edge_block(be, h2), _stats_spec(h2)],
        out_shape=[jax.ShapeDtypeStruct((e, h2), bf16),
                   jax.ShapeDtypeStruct((8, h2), f32)],
    )(n2e, edge_rep, wt0[:h], wt0[h:])
    sc1, sh1 = _finalize_stats(st1, e, g0a, b0a)

    y2, st2 = pl.pallas_call(
        _p2_body,
        grid=grid,
        in_specs=[_edge_block(be, h2), _full_block((1, h2)), _full_block((1, h2)),
                  _full_block((h2, h))],
        out_specs=[_edge_block(be, h), _stats_spec(h)],
        out_shape=[jax.ShapeDtypeStruct((e, h), bf16),
                   jax.ShapeDtypeStruct((8, h), f32)],
    )(y1, sc1, sh1, W0b.T.astype(bf16))
    sc2, sh2 = _finalize_stats(st2, e, g0b, b0b)

    c1 = (1.0 + eps1).reshape(1, 1)
    y3, y5, st3, st5 = pl.pallas_call(
        _p3_body,
        grid=grid,
        in_specs=[_edge_block(be, h), _edge_block(be, h), _full_block((1, h)),
                  _full_block((1, h)), _full_block((1, 1)),
                  _full_block((h, h2)), _full_block((h, h2))],
        out_specs=[_edge_block(be, h2), _edge_block(be, h2),
                   _stats_spec(h2), _stats_spec(h2)],
        out_shape=[jax.ShapeDtypeStruct((e, h2), bf16),
                   jax.ShapeDtypeStruct((e, h2), bf16),
                   jax.ShapeDtypeStruct((8, h2), f32),
                   jax.ShapeDtypeStruct((8, h2), f32)],
    )(y2, edge_rep, sc2, sh2, c1, W1a.T.astype(bf16), W2a.T.astype(bf16))
    sc3, sh3 = _finalize_stats(st3, e, g1a, b1a)
    sc5, sh5 = _finalize_stats(st5, e, g2a, b2a)

    y4, y6, st4, st6 = pl.pallas_call(
        _p4_body,
        grid=grid,
        in_specs=[_edge_block(be, h2), _edge_block(be, h2),
                  _full_block((1, h2)), _full_block((1, h2)),
                  _full_block((1, h2)), _full_block((1, h2)),
                  _full_block((h2, h)), _full_block((h2, h))],
        out_specs=[_edge_block(be, h), _edge_block(be, h),
                   _stats_spec(h), _stats_spec(h)],
        out_shape=[jax.ShapeDtypeStruct((e, h), bf16),
                   jax.ShapeDtypeStruct((e, h), bf16),
                   jax.ShapeDtypeStruct((8, h), f32),
                   jax.ShapeDtypeStruct((8, h), f32)],
    )(y3, y5, sc3, sh3, sc5, sh5, W1b.T.astype(bf16), W2b.T.astype(bf16))
    sc4, sh4 = _finalize_stats(st4, e, g1b, b1b)
    sc6, sh6 = _finalize_stats(st6, e, g2b, b2b)

    edge_out, e2n = pl.pallas_call(
        _p5_body,
        grid=grid,
        in_specs=[_edge_block(be, h), _edge_block(be, h),
                  _full_block((1, h)), _full_block((1, h)),
                  _full_block((1, h)), _full_block((1, h))],
        out_specs=[_edge_block(be, h), _edge_block(be, h)],
        out_shape=[jax.ShapeDtypeStruct((e, h), f32),
                   jax.ShapeDtypeStruct((e, h), f32)],
    )(y4, y6, sc4, sh4, sc6, sh6)

    parts = _make_sc_scatter(n, h, e, nc, ns, s_sc, c_sc)(
        e2n, src3, dst3, jnp.zeros((n, h), f32))

    coef = (1.0 + eps2[0] - degree).reshape(n, 1)
    node_out = pl.pallas_call(
        _node_body,
        grid=(1,),
        in_specs=[_full_block((n, h)), _full_block((n, 1)), _full_block((n, h)),
                  _full_block((n, h)),
                  _full_block((h, h2)), _full_block((1, h2)), _full_block((1, h2)),
                  _full_block((h2, h)), _full_block((1, h)), _full_block((1, h))],
        out_specs=_full_block((n, h)),
        out_shape=jax.ShapeDtypeStruct((n, h), f32),
    )(node_rep, coef, parts[0], parts[1], Wna.T, gna.reshape(1, -1),
      bna.reshape(1, -1), Wnb.T, gnb.reshape(1, -1), bnb.reshape(1, -1))

    return (node_out, edge_out)


# speedup vs baseline: 1.0284x; 1.0284x over previous
import functools

import jax
import jax.numpy as jnp
from jax import lax
from jax.experimental import pallas as pl
from jax.experimental.pallas import tpu as pltpu
from jax.experimental.pallas import tpu_sc as plsc

_BN_EPS = 1e-5


def _sc_dims():
    try:
        info = plsc.get_sparse_core_info()
        return info.num_cores, info.num_subcores
    except Exception:
        return 2, 16


def _make_sc_gather(n, h, e, nc, ns, s, csteps):
    nw = nc * ns
    per_w = e // nw
    steps = per_w // s
    chunks = steps // csteps
    nbuf = 5
    assert chunks * csteps == steps and csteps % nbuf == 0
    mesh = plsc.VectorSubcoreMesh(core_axis_name="c", subcore_axis_name="s")

    @functools.partial(
        pl.kernel,
        mesh=mesh,
        out_type=jax.ShapeDtypeStruct((e, h), jnp.float32),
        scratch_types=(
            [pltpu.VMEM((csteps, s), jnp.int32)] * 2
            + [pltpu.VMEM((s, h), jnp.float32)] * nbuf
            + [pltpu.SemaphoreType.DMA] * (2 * nbuf)
        ),
    )
    def gather_kernel(node_hbm, src_hbm, dst_hbm, out_hbm, src_v, dst_v,
                      *bufs):
        rows = bufs[:nbuf]
        sg = bufs[nbuf:2 * nbuf]
        ss = bufs[2 * nbuf:]
        wid = lax.axis_index("s") * nc + lax.axis_index("c")
        base = wid * per_w

        def drain_store(b):
            pltpu.make_async_copy(rows[b], out_hbm.at[pl.ds(base, s)],
                                  ss[b]).wait()

        def chunk_body(ch, carry):
            pltpu.sync_copy(src_hbm.at[wid, ch], src_v)
            pltpu.sync_copy(dst_hbm.at[wid, ch], dst_v)
            cbase = ch * csteps

            def body(t, carry2):
                i = nbuf * t
                cps = []
                for b in range(nbuf):
                    @pl.when(t > 0)
                    def _(b=b):
                        drain_store(b)

                    cps.append(pltpu.async_copy(
                        node_hbm.at[src_v.at[i + b]], rows[b], sg[b]))
                dps = []
                for b in range(nbuf):
                    cps[b].wait()
                    dps.append(pltpu.async_copy(
                        node_hbm.at[dst_v.at[i + b]], rows[b], sg[b],
                        add=True))
                for b in range(nbuf):
                    dps[b].wait()
                    pltpu.async_copy(
                        rows[b],
                        out_hbm.at[pl.ds(base + (cbase + i + b) * s, s)],
                        ss[b])
                return carry2

            lax.fori_loop(0, csteps // nbuf, body, 0)
            for b in range(nbuf):
                drain_store(b)
            return carry

        lax.fori_loop(0, chunks, chunk_body, 0)

    return gather_kernel


def _make_sc_scatter(n, h, e, nc, ns, s, csteps):
    nw = nc * ns
    per_w = e // nw
    steps = per_w // s
    chunks = steps // csteps
    nbuf = 5
    assert chunks * csteps == steps and csteps % nbuf == 0
    rpt = (n // ns) // 8 * 8
    tail = n - rpt * ns
    mesh = plsc.VectorSubcoreMesh(core_axis_name="c", subcore_axis_name="s")

    @functools.partial(
        pl.kernel,
        mesh=mesh,
        out_type=jax.ShapeDtypeStruct((nc, n, h), jnp.float32),
        scratch_types=(
            [pltpu.VMEM((csteps, s), jnp.int32)] * 2
            + [pltpu.VMEM((s, h), jnp.float32)] * nbuf
            + [pltpu.VMEM_SHARED((n, h), jnp.float32)]
            + [pltpu.SemaphoreType.DMA] * (2 * nbuf)
        ),
    )
    def scatter_kernel(e2n_hbm, src_hbm, dst_hbm, zeros_hbm, out_hbm,
                       src_v, dst_v, *bufs):
        rows = bufs[:nbuf]
        agg_sh = bufs[nbuf]
        lg = bufs[nbuf + 1:2 * nbuf + 1]
        sc = bufs[2 * nbuf + 1:]
        cid = lax.axis_index("c")
        sid = lax.axis_index("s")
        wid = sid * nc + cid
        base = wid * per_w
        pltpu.sync_copy(zeros_hbm.at[pl.ds(sid * rpt, rpt)],
                        agg_sh.at[pl.ds(sid * rpt, rpt)])

        @pl.when(sid == ns - 1)
        def _():
            pltpu.sync_copy(zeros_hbm.at[pl.ds(rpt * ns, tail)],
                            agg_sh.at[pl.ds(rpt * ns, tail)])

        plsc.subcore_barrier()

        def load(gi, b):
            return pltpu.async_copy(e2n_hbm.at[pl.ds(base + gi * s, s)],
                                    rows[b], lg[b])

        def chunk_body(ch, carry):
            pltpu.sync_copy(src_hbm.at[wid, ch], src_v)
            pltpu.sync_copy(dst_hbm.at[wid, ch], dst_v)
            cbase = ch * csteps

            def body(t, carry2):
                i = nbuf * t
                cps = [load(cbase + i + b, b) for b in range(nbuf)]
                sps = []
                for b in range(nbuf):
                    cps[b].wait()
                    sps.append(pltpu.async_copy(
                        rows[b], agg_sh.at[src_v.at[i + b]], sc[b],
                        add=True))
                    sps.append(pltpu.async_copy(
                        rows[b], agg_sh.at[dst_v.at[i + b]], sc[b],
                        add=True))
                for p in sps:
                    p.wait()
                return carry2

            lax.fori_loop(0, csteps // nbuf, body, 0)
            return carry

        lax.fori_loop(0, chunks, chunk_body, 0)
        plsc.subcore_barrier()
        pltpu.sync_copy(agg_sh.at[pl.ds(sid * rpt, rpt)],
                        out_hbm.at[cid, pl.ds(sid * rpt, rpt)])

        @pl.when(sid == ns - 1)
        def _():
            pltpu.sync_copy(agg_sh.at[pl.ds(rpt * ns, tail)],
                            out_hbm.at[cid, pl.ds(rpt * ns, tail)])

    return scatter_kernel


def _finalize_stats(st, count, g, b):
    mean = st[0] / count
    var = st[1] / count - mean * mean
    scale = g * jax.lax.rsqrt(var + _BN_EPS)
    shift = b - mean * scale
    return scale.reshape(1, -1), shift.reshape(1, -1)


def _accum_stats(stats_ref, y, j):
    @pl.when(j == 0)
    def _():
        stats_ref[...] = jnp.zeros_like(stats_ref)

    stats_ref[0:1, :] += jnp.sum(y, axis=0, keepdims=True)
    stats_ref[1:2, :] += jnp.sum(y * y, axis=0, keepdims=True)


def _p1_body(n2e_ref, er_ref, wt_top_ref, wt_bot_ref, y1_ref, st_ref):
    j = pl.program_id(0)
    y = (jnp.dot(n2e_ref[...].astype(jnp.bfloat16), wt_top_ref[...],
                 preferred_element_type=jnp.float32)
         + jnp.dot(er_ref[...].astype(jnp.bfloat16), wt_bot_ref[...],
                   preferred_element_type=jnp.float32))
    y1_ref[...] = y.astype(jnp.bfloat16)
    _accum_stats(st_ref, y, j)


def _p2_body(y1_ref, sc_ref, sh_ref, wt_ref, y2_ref, st_ref):
    j = pl.program_id(0)
    h = jnp.maximum(
        y1_ref[...].astype(jnp.float32) * sc_ref[...] + sh_ref[...], 0.0)
    y = jnp.dot(h.astype(jnp.bfloat16), wt_ref[...],
                preferred_element_type=jnp.float32)
    y2_ref[...] = y.astype(jnp.bfloat16)
    _accum_stats(st_ref, y, j)


def _p3_body(y2_ref, er_ref, sc_ref, sh_ref, c1_ref, wt1_ref, wt2_ref,
             y3_ref, y5_ref, st3_ref, st5_ref):
    j = pl.program_id(0)
    m = jnp.maximum(
        y2_ref[...].astype(jnp.float32) * sc_ref[...] + sh_ref[...], 0.0)
    en = (c1_ref[0, 0] * er_ref[...] + m).astype(jnp.bfloat16)
    y3 = jnp.dot(en, wt1_ref[...], preferred_element_type=jnp.float32)
    y5 = jnp.dot(en, wt2_ref[...], preferred_element_type=jnp.float32)
    y3_ref[...] = y3.astype(jnp.bfloat16)
    y5_ref[...] = y5.astype(jnp.bfloat16)
    _accum_stats(st3_ref, y3, j)
    _accum_stats(st5_ref, y5, j)


def _p4_body(y3_ref, y5_ref, sc3_ref, sh3_ref, sc5_ref, sh5_ref,
             wt1_ref, wt2_ref, y4_ref, y6_ref, st4_ref, st6_ref):
    j = pl.program_id(0)
    h3 = jnp.maximum(
        y3_ref[...].astype(jnp.float32) * sc3_ref[...] + sh3_ref[...], 0.0)
    h5 = jnp.maximum(
        y5_ref[...].astype(jnp.float32) * sc5_ref[...] + sh5_ref[...], 0.0)
    y4 = jnp.dot(h3.astype(jnp.bfloat16), wt1_ref[...],
                 preferred_element_type=jnp.float32)
    y6 = jnp.dot(h5.astype(jnp.bfloat16), wt2_ref[...],
                 preferred_element_type=jnp.float32)
    y4_ref[...] = y4.astype(jnp.bfloat16)
    y6_ref[...] = y6.astype(jnp.bfloat16)
    _accum_stats(st4_ref, y4, j)
    _accum_stats(st6_ref, y6, j)


def _p5_body(y4_ref, y6_ref, sc4_ref, sh4_ref, sc6_ref, sh6_ref,
             eo_ref, e2n_ref):
    eo_ref[...] = jnp.maximum(
        y4_ref[...].astype(jnp.float32) * sc4_ref[...] + sh4_ref[...], 0.0)
    e2n_ref[...] = jnp.maximum(
        y6_ref[...].astype(jnp.float32) * sc6_ref[...] + sh6_ref[...], 0.0)


def _node_body(nr_ref, coef_ref, agg0_ref, agg1_ref, wa_ref, ga_ref, ba_ref,
               wb_ref, gb_ref, bb_ref, out_ref):
    x = coef_ref[...] * nr_ref[...] + (agg0_ref[...] + agg1_ref[...])
    y = jnp.dot(x, wa_ref[...], preferred_element_type=jnp.float32)
    m = jnp.mean(y, axis=0, keepdims=True)
    v = jnp.mean(y * y, axis=0, keepdims=True) - m * m
    h = jnp.maximum((y - m) * jax.lax.rsqrt(v + _BN_EPS) * ga_ref[...]
                    + ba_ref[...], 0.0)
    y2 = jnp.dot(h, wb_ref[...], preferred_element_type=jnp.float32)
    m2 = jnp.mean(y2, axis=0, keepdims=True)
    v2 = jnp.mean(y2 * y2, axis=0, keepdims=True) - m2 * m2
    out_ref[...] = jnp.maximum(
        (y2 - m2) * jax.lax.rsqrt(v2 + _BN_EPS) * gb_ref[...] + bb_ref[...],
        0.0)


def _edge_block(be, c):
    return pl.BlockSpec((be, c), lambda j: (j, 0))


def _full_block(shape):
    return pl.BlockSpec(shape, lambda j: tuple(0 for _ in shape))


def _stats_spec(c):
    return pl.BlockSpec((8, c), lambda j: (0, 0))


def kernel(node_rep, edge_rep, edge_index, degree, W0a, g0a, b0a, W0b, g0b,
           b0b, W1a, g1a, b1a, W1b, g1b, b1b, W2a, g2a, b2a, W2b, g2b, b2b,
           Wna, gna, bna, Wnb, gnb, bnb, eps1, eps2):
    n, h = node_rep.shape
    e = edge_rep.shape[0]
    h2 = 2 * h
    be = 8000
    nb = e // be
    assert nb * be == e
    f32 = jnp.float32

    src = edge_index[0]
    dst = edge_index[1]
    nc, ns = _sc_dims()
    nw = nc * ns
    bf16 = jnp.bfloat16
    s_sc = 40
    c_sc = 50
    n_ch = e // nw // s_sc // c_sc
    src3 = src.reshape(nw, n_ch, c_sc, s_sc)
    dst3 = dst.reshape(nw, n_ch, c_sc, s_sc)
    n2e = _make_sc_gather(n, h, e, nc, ns, s_sc, c_sc)(node_rep, src3, dst3)

    grid = (nb,)

    wt0 = W0a.T.astype(bf16)
    y1, st1 = pl.pallas_call(
        _p1_body,
        grid=grid,
        in_specs=[_edge_block(be, h), _edge_block(be, h),
                  _full_block((h, h2)), _full_block((h, h2))],
        out_specs=[_edge_block(be, h2), _stats_spec(h2)],
        out_shape=[jax.ShapeDtypeStruct((e, h2), bf16),
                   jax.ShapeDtypeStruct((8, h2), f32)],
    )(n2e, edge_rep, wt0[:h], wt0[h:])
    sc1, sh1 = _finalize_stats(st1, e, g0a, b0a)

    y2, st2 = pl.pallas_call(
        _p2_body,
        grid=grid,
        in_specs=[_edge_block(be, h2), _full_block((1, h2)), _full_block((1, h2)),
                  _full_block((h2, h))],
        out_specs=[_edge_block(be, h), _stats_spec(h)],
        out_shape=[jax.ShapeDtypeStruct((e, h), bf16),
                   jax.ShapeDtypeStruct((8, h), f32)],
    )(y1, sc1, sh1, W0b.T.astype(bf16))
    sc2, sh2 = _finalize_stats(st2, e, g0b, b0b)

    c1 = (1.0 + eps1).reshape(1, 1)
    y3, y5, st3, st5 = pl.pallas_call(
        _p3_body,
        grid=grid,
        in_specs=[_edge_block(be, h), _edge_block(be, h), _full_block((1, h)),
                  _full_block((1, h)), _full_block((1, 1)),
                  _full_block((h, h2)), _full_block((h, h2))],
        out_specs=[_edge_block(be, h2), _edge_block(be, h2),
                   _stats_spec(h2), _stats_spec(h2)],
        out_shape=[jax.ShapeDtypeStruct((e, h2), bf16),
                   jax.ShapeDtypeStruct((e, h2), bf16),
                   jax.ShapeDtypeStruct((8, h2), f32),
                   jax.ShapeDtypeStruct((8, h2), f32)],
    )(y2, edge_rep, sc2, sh2, c1, W1a.T.astype(bf16), W2a.T.astype(bf16))
    sc3, sh3 = _finalize_stats(st3, e, g1a, b1a)
    sc5, sh5 = _finalize_stats(st5, e, g2a, b2a)

    y4, y6, st4, st6 = pl.pallas_call(
        _p4_body,
        grid=grid,
        in_specs=[_edge_block(be, h2), _edge_block(be, h2),
                  _full_block((1, h2)), _full_block((1, h2)),
                  _full_block((1, h2)), _full_block((1, h2)),
                  _full_block((h2, h)), _full_block((h2, h))],
        out_specs=[_edge_block(be, h), _edge_block(be, h),
                   _stats_spec(h), _stats_spec(h)],
        out_shape=[jax.ShapeDtypeStruct((e, h), bf16),
                   jax.ShapeDtypeStruct((e, h), bf16),
                   jax.ShapeDtypeStruct((8, h), f32),
                   jax.ShapeDtypeStruct((8, h), f32)],
    )(y3, y5, sc3, sh3, sc5, sh5, W1b.T.astype(bf16), W2b.T.astype(bf16))
    sc4, sh4 = _finalize_stats(st4, e, g1b, b1b)
    sc6, sh6 = _finalize_stats(st6, e, g2b, b2b)

    edge_out, e2n = pl.pallas_call(
        _p5_body,
        grid=grid,
        in_specs=[_edge_block(be, h), _edge_block(be, h),
                  _full_block((1, h)), _full_block((1, h)),
                  _full_block((1, h)), _full_block((1, h))],
        out_specs=[_edge_block(be, h), _edge_block(be, h)],
        out_shape=[jax.ShapeDtypeStruct((e, h), f32),
                   jax.ShapeDtypeStruct((e, h), f32)],
    )(y4, y6, sc4, sh4, sc6, sh6)

    parts = _make_sc_scatter(n, h, e, nc, ns, s_sc, c_sc)(
        e2n, src3, dst3, jnp.zeros((n, h), f32))

    coef = (1.0 + eps2[0] - degree).reshape(n, 1)
    node_out = pl.pallas_call(
        _node_body,
        grid=(1,),
        in_specs=[_full_block((n, h)), _full_block((n, 1)), _full_block((n, h)),
                  _full_block((n, h)),
                  _full_block((h, h2)), _full_block((1, h2)), _full_block((1, h2)),
                  _full_block((h2, h)), _full_block((1, h)), _full_block((1, h))],
        out_specs=_full_block((n, h)),
        out_shape=jax.ShapeDtypeStruct((n, h), f32),
    )(node_rep, coef, parts[0], parts[1], Wna.T, gna.reshape(1, -1),
      bna.reshape(1, -1), Wnb.T, gnb.reshape(1, -1), bnb.reshape(1, -1))

    return (node_out, edge_out)
